# baseline (speedup 1.0000x reference)
import functools

import jax
import jax.numpy as jnp
from jax import lax
from jax.experimental import pallas as pl
from jax.experimental.pallas import tpu as pltpu
from jax.experimental.pallas import tpu_sc as plsc

DT = 0.02
NC = 2
NS = 16
NW = NC * NS
L = 16
CHUNK_A = 4000
CHUNK_B = 1600
NBUF = 3
UNROLL = 4


def _mesh():
    return plsc.VectorSubcoreMesh(core_axis_name="c", subcore_axis_name="s")


def _sc_params():
    return pltpu.CompilerParams(needs_layout_passes=False)


@functools.lru_cache(maxsize=None)
def _build_gather(n_nodes, n_edges):
    e_per_w = n_edges // NW
    half_per_w = e_per_w // 2
    chunk = CHUNK_A
    half = chunk // 2
    assert n_edges % (NW * chunk) == 0
    n_chunks = e_per_w // chunk
    n_groups = n_chunks // NBUF
    n_rem = n_chunks % NBUF
    assert n_chunks >= 2 * NBUF
    assert chunk % (2 * L) == 0 and chunk % 8 == 0

    scratch = [pltpu.VMEM((n_nodes,), jnp.float32)]
    scratch += [pltpu.VMEM((chunk,), jnp.int32) for _ in range(NBUF)]
    scratch += [pltpu.VMEM((half,), jnp.int32) for _ in range(NBUF)]
    scratch += [pltpu.SemaphoreType.DMA for _ in range(2 * NBUF)]

    @functools.partial(
        pl.kernel,
        out_type=jax.ShapeDtypeStruct((NW * half_per_w,), jnp.int32),
        mesh=_mesh(),
        scratch_types=scratch,
        compiler_params=_sc_params(),
    )
    def gather_k(x_hbm, src_hbm, stash_hbm, table_v, *rest):
        src_v = rest[0:NBUF]
        pk_v = rest[NBUF:2 * NBUF]
        in_sems = rest[2 * NBUF:3 * NBUF]
        out_sems = rest[3 * NBUF:4 * NBUF]

        wid = lax.axis_index("s") * NC + lax.axis_index("c")
        base = wid * e_per_w
        stash_base = wid * half_per_w

        def in_desc(b, c):
            s_all = pl.ds(base + c * chunk, chunk)
            return pltpu.make_async_copy(src_hbm.at[s_all], src_v[b],
                                         in_sems[b])

        def out_desc(b, c):
            s_st = pl.ds(stash_base + c * half, half)
            return pltpu.make_async_copy(pk_v[b], stash_hbm.at[s_st],
                                         out_sems[b])

        def compute(b):
            sb, pb = src_v[b], pk_v[b]

            def vec_body(j, _):
                s0 = pl.ds(2 * j * L, L)
                s1 = pl.ds((2 * j + 1) * L, L)
                v0 = jnp.maximum(plsc.load_gather(table_v, [sb[s0]]), 0.0)
                v1 = jnp.maximum(plsc.load_gather(table_v, [sb[s1]]), 0.0)
                packed = plsc.pack(v0, v1, format=plsc.PackFormat.INTERLEAVED)
                pb[pl.ds(j * L, L)] = plsc.bitcast(packed, jnp.int32)
                return _

            lax.fori_loop(0, chunk // (2 * L), vec_body, None, unroll=UNROLL)

        for b in range(NBUF):
            in_desc(b, b).start()
        pltpu.sync_copy(x_hbm, table_v)

        for b in range(NBUF):
            in_desc(b, b).wait()
            compute(b)
            out_desc(b, b).start()
            in_desc(b, b + NBUF).start()

        def main(g, _):
            for b in range(NBUF):
                c = g * NBUF + b
                in_desc(b, c).wait()
                out_desc(b, c - NBUF).wait()
                compute(b)
                out_desc(b, c).start()

                @pl.when(c + NBUF < n_chunks)
                def _next(b=b, c=c):
                    in_desc(b, c + NBUF).start()
            return _

        lax.fori_loop(1, n_groups, main, None)

        for b in range(n_rem):
            c = n_groups * NBUF + b
            in_desc(b, c).wait()
            out_desc(b, c - NBUF).wait()
            compute(b)
            out_desc(b, c).start()

        for k in range(NBUF):
            c = n_chunks - NBUF + k
            out_desc(c % NBUF, c).wait()

    return gather_k


@functools.lru_cache(maxsize=None)
def _build_scatter(n_nodes, n_edges):
    e_per_w = n_edges // NW
    half_per_w = e_per_w // 2
    chunk = CHUNK_B
    half = chunk // 2
    assert n_edges % (NW * chunk) == 0
    n_chunks = e_per_w // chunk
    n_groups = n_chunks // NBUF
    n_rem = n_chunks % NBUF
    assert n_chunks >= 2 * NBUF
    assert chunk % (2 * L) == 0 and chunk % 8 == 0
    assert n_nodes % L == 0

    scratch = [pltpu.VMEM((n_nodes,), jnp.float32)]
    scratch += [pltpu.VMEM((chunk,), jnp.int32) for _ in range(NBUF)]
    scratch += [pltpu.VMEM((chunk,), jnp.float32) for _ in range(NBUF)]
    scratch += [pltpu.VMEM((half,), jnp.int32) for _ in range(NBUF)]
    scratch += [pltpu.SemaphoreType.DMA for _ in range(NBUF)]

    @functools.partial(
        pl.kernel,
        out_type=jax.ShapeDtypeStruct((NW * n_nodes,), jnp.float32),
        mesh=_mesh(),
        scratch_types=scratch,
        compiler_params=_sc_params(),
    )
    def scatter_k(tgt_hbm, w_hbm, stash_hbm, part_hbm, acc_v, *rest):
        tgt_v = rest[0:NBUF]
        w_v = rest[NBUF:2 * NBUF]
        pk_v = rest[2 * NBUF:3 * NBUF]
        in_sems = rest[3 * NBUF:4 * NBUF]

        wid = lax.axis_index("s") * NC + lax.axis_index("c")
        base = wid * e_per_w
        stash_base = wid * half_per_w

        def in_descs(b, c):
            s_all = pl.ds(base + c * chunk, chunk)
            s_st = pl.ds(stash_base + c * half, half)
            return (
                pltpu.make_async_copy(tgt_hbm.at[s_all], tgt_v[b], in_sems[b]),
                pltpu.make_async_copy(w_hbm.at[s_all], w_v[b], in_sems[b]),
                pltpu.make_async_copy(stash_hbm.at[s_st], pk_v[b], in_sems[b]),
            )

        def in_start(b, c):
            for d in in_descs(b, c):
                d.start()

        def in_wait(b, c):
            for d in in_descs(b, c):
                d.wait()

        def compute(b):
            tb, wb, pb = tgt_v[b], w_v[b], pk_v[b]

            def vec_body(j, _):
                s0 = pl.ds(2 * j * L, L)
                s1 = pl.ds((2 * j + 1) * L, L)
                packed = plsc.bitcast(pb[pl.ds(j * L, L)], jnp.bfloat16)
                v0, v1 = plsc.unpack(packed, format=plsc.PackFormat.INTERLEAVED)
                plsc.addupdate_scatter(acc_v, [tb[s0]], v0 * wb[s0])
                plsc.addupdate_scatter(acc_v, [tb[s1]], v1 * wb[s1])
                return _

            lax.fori_loop(0, chunk // (2 * L), vec_body, None, unroll=UNROLL)

        for b in range(NBUF):
            in_start(b, b)

        zeros = jnp.zeros((L,), jnp.float32)

        def zero_body(i, _):
            acc_v[pl.ds(i * L, L)] = zeros
            return _

        lax.fori_loop(0, n_nodes // L, zero_body, None, unroll=8)

        def main(g, _):
            for b in range(NBUF):
                c = g * NBUF + b
                in_wait(b, c)
                compute(b)

                @pl.when(c + NBUF < n_chunks)
                def _next(b=b, c=c):
                    in_start(b, c + NBUF)
            return _

        lax.fori_loop(0, n_groups, main, None)

        for b in range(n_rem):
            c = n_groups * NBUF + b
            in_wait(b, c)
            compute(b)

        pltpu.sync_copy(acc_v, part_hbm.at[pl.ds(wid * n_nodes, n_nodes)])

    return scatter_k


def _weight_body(sign_ref, cnt_ref, str_ref, o_ref):
    o_ref[...] = sign_ref[...] * cnt_ref[...] * jnp.maximum(str_ref[...], 0.0)


def _epilogue_body(x_ref, bias_ref, tau_ref, part_ref, o_ref):
    summed = jnp.sum(part_ref[...], axis=0)
    x = x_ref[...]
    o_ref[...] = x + DT * ((-x + bias_ref[...] + summed) / tau_ref[...])


def kernel(x, source_indices, target_indices, sign, syn_count, syn_strength,
           bias, time_const):
    n_nodes = x.shape[0]
    n_edges = source_indices.shape[0]

    w_cols = 128
    w_rows = n_edges // w_cols
    w_blk = 5000
    assert n_edges % w_cols == 0 and w_rows % w_blk == 0
    spec = pl.BlockSpec((w_blk, w_cols), lambda i: (i, 0))

    gather_k = _build_gather(n_nodes, n_edges)
    stash = gather_k(x, source_indices.astype(jnp.int32))

    w = pl.pallas_call(
        _weight_body,
        grid=(w_rows // w_blk,),
        in_specs=[spec, spec, spec],
        out_specs=spec,
        out_shape=jax.ShapeDtypeStruct((w_rows, w_cols), jnp.float32),
    )(sign.reshape(w_rows, w_cols), syn_count.reshape(w_rows, w_cols),
      syn_strength.reshape(w_rows, w_cols))
    w = w.reshape(n_edges)

    scatter_k = _build_scatter(n_nodes, n_edges)
    partials = scatter_k(target_indices.astype(jnp.int32), w, stash)
    partials = partials.reshape(NW, n_nodes)

    x_new = pl.pallas_call(
        _epilogue_body,
        out_shape=jax.ShapeDtypeStruct((n_nodes,), jnp.float32),
    )(x, bias, time_const, partials)
    return x_new


# speedup vs baseline: 1.0335x; 1.0335x over previous
import functools

import jax
import jax.numpy as jnp
from jax import lax
from jax.experimental import pallas as pl
from jax.experimental.pallas import tpu as pltpu
from jax.experimental.pallas import tpu_sc as plsc

DT = 0.02
NC = 2
NS = 16
NW = NC * NS
L = 16
CHUNK = 1600
HALF = CHUNK // 2
NBUF = 3
UNROLL = 4


def _mesh():
    return plsc.VectorSubcoreMesh(core_axis_name="c", subcore_axis_name="s")


def _sc_params():
    return pltpu.CompilerParams(needs_layout_passes=False)


@functools.lru_cache(maxsize=None)
def _build_fused(n_nodes, n_edges):
    assert n_edges % (NW * CHUNK) == 0
    e_per_w = n_edges // NW
    n_chunks = e_per_w // CHUNK
    n_groups = n_chunks // NBUF
    n_rem = n_chunks % NBUF
    assert n_chunks >= 2 * NBUF
    assert CHUNK % (2 * L) == 0 and CHUNK % 8 == 0
    assert n_nodes % L == 0
    half_per_w = e_per_w // 2

    vmem = [
        pltpu.VMEM((n_nodes,), jnp.float32),
    ]
    vmem += [pltpu.VMEM((CHUNK,), jnp.int32) for _ in range(NBUF)]
    vmem += [pltpu.VMEM((CHUNK,), jnp.float32) for _ in range(NBUF)]
    vmem += [pltpu.VMEM((CHUNK,), jnp.float32) for _ in range(NBUF)]
    vmem += [pltpu.VMEM((CHUNK,), jnp.float32) for _ in range(NBUF)]
    vmem += [pltpu.VMEM((CHUNK,), jnp.int32) for _ in range(NBUF)]
    vmem += [pltpu.VMEM((HALF,), jnp.int32) for _ in range(NBUF)]
    scratch = vmem + [
        pltpu.HBM((NW * half_per_w,), jnp.int32),
    ]
    scratch += [pltpu.SemaphoreType.DMA for _ in range(3 * NBUF)]

    @functools.partial(
        pl.kernel,
        out_type=jax.ShapeDtypeStruct((NW * n_nodes,), jnp.float32),
        mesh=_mesh(),
        scratch_types=scratch,
        compiler_params=_sc_params(),
    )
    def fused(x_hbm, src_hbm, sign_hbm, cnt_hbm, str_hbm, tgt_hbm, part_hbm,
              work_v, *rest):
        src_v = rest[0:NBUF]
        sign_v = rest[NBUF:2 * NBUF]
        cnt_v = rest[2 * NBUF:3 * NBUF]
        str_v = rest[3 * NBUF:4 * NBUF]
        tgt_v = rest[4 * NBUF:5 * NBUF]
        pk_v = rest[5 * NBUF:6 * NBUF]
        stash = rest[6 * NBUF]
        ina_sems = rest[6 * NBUF + 1:6 * NBUF + 1 + NBUF]
        outa_sems = rest[6 * NBUF + 1 + NBUF:6 * NBUF + 1 + 2 * NBUF]
        inb_sems = rest[6 * NBUF + 1 + 2 * NBUF:6 * NBUF + 1 + 3 * NBUF]

        cid = lax.axis_index("c")
        sid = lax.axis_index("s")
        wid = sid * NC + cid
        base = wid * e_per_w
        stash_base = wid * half_per_w


        def ina_descs(b, c):
            s_all = pl.ds(base + c * CHUNK, CHUNK)
            return (
                pltpu.make_async_copy(src_hbm.at[s_all], src_v[b], ina_sems[b]),
                pltpu.make_async_copy(sign_hbm.at[s_all], sign_v[b], ina_sems[b]),
                pltpu.make_async_copy(cnt_hbm.at[s_all], cnt_v[b], ina_sems[b]),
                pltpu.make_async_copy(str_hbm.at[s_all], str_v[b], ina_sems[b]),
            )

        def outa_desc(b, c):
            s_st = pl.ds(stash_base + c * HALF, HALF)
            return pltpu.make_async_copy(pk_v[b], stash.at[s_st], outa_sems[b])

        def ina_start(b, c):
            for d in ina_descs(b, c):
                d.start()

        def ina_wait(b, c):
            for d in ina_descs(b, c):
                d.wait()

        def compute_a(b):
            sb, gb, cb, tb, pb = (src_v[b], sign_v[b], cnt_v[b], str_v[b],
                                  pk_v[b])

            def vec_body(j, _):
                s0 = pl.ds(2 * j * L, L)
                s1 = pl.ds((2 * j + 1) * L, L)
                v0 = jnp.maximum(plsc.load_gather(work_v, [sb[s0]]), 0.0)
                v1 = jnp.maximum(plsc.load_gather(work_v, [sb[s1]]), 0.0)
                c0 = v0 * (gb[s0] * cb[s0] * jnp.maximum(tb[s0], 0.0))
                c1 = v1 * (gb[s1] * cb[s1] * jnp.maximum(tb[s1], 0.0))
                packed = plsc.pack(c0, c1, format=plsc.PackFormat.INTERLEAVED)
                pb[pl.ds(j * L, L)] = plsc.bitcast(packed, jnp.int32)
                return _

            lax.fori_loop(0, CHUNK // (2 * L), vec_body, None, unroll=UNROLL)

        for b in range(NBUF):
            ina_start(b, b)
        pltpu.sync_copy(x_hbm, work_v)

        for b in range(NBUF):
            ina_wait(b, b)
            compute_a(b)
            outa_desc(b, b).start()
            ina_start(b, b + NBUF)

        def main_a(g, _):
            for b in range(NBUF):
                c = g * NBUF + b
                ina_wait(b, c)
                outa_desc(b, c - NBUF).wait()
                compute_a(b)
                outa_desc(b, c).start()

                @pl.when(c + NBUF < n_chunks)
                def _next(b=b, c=c):
                    ina_start(b, c + NBUF)
            return _

        lax.fori_loop(1, n_groups, main_a, None)

        for b in range(n_rem):
            c = n_groups * NBUF + b
            ina_wait(b, c)
            outa_desc(b, c - NBUF).wait()
            compute_a(b)
            outa_desc(b, c).start()

        for k in range(NBUF):
            c = n_chunks - NBUF + k
            outa_desc(c % NBUF, c).wait()


        def inb_descs(b, c):
            s_all = pl.ds(base + c * CHUNK, CHUNK)
            s_st = pl.ds(stash_base + c * HALF, HALF)
            return (
                pltpu.make_async_copy(tgt_hbm.at[s_all], tgt_v[b], inb_sems[b]),
                pltpu.make_async_copy(stash.at[s_st], pk_v[b], inb_sems[b]),
            )

        def inb_start(b, c):
            for d in inb_descs(b, c):
                d.start()

        def inb_wait(b, c):
            for d in inb_descs(b, c):
                d.wait()

        def compute_b(b):
            tb, pb = tgt_v[b], pk_v[b]

            def vec_body(j, _):
                s0 = pl.ds(2 * j * L, L)
                s1 = pl.ds((2 * j + 1) * L, L)
                packed = plsc.bitcast(pb[pl.ds(j * L, L)], jnp.bfloat16)
                c0, c1 = plsc.unpack(packed, format=plsc.PackFormat.INTERLEAVED)
                plsc.addupdate_scatter(work_v, [tb[s0]], c0)
                plsc.addupdate_scatter(work_v, [tb[s1]], c1)
                return _

            lax.fori_loop(0, CHUNK // (2 * L), vec_body, None, unroll=UNROLL)

        for b in range(NBUF):
            inb_start(b, b)

        zeros = jnp.zeros((L,), jnp.float32)

        def zero_body(i, _):
            work_v[pl.ds(i * L, L)] = zeros
            return _

        lax.fori_loop(0, n_nodes // L, zero_body, None, unroll=8)

        def main_b(g, _):
            for b in range(NBUF):
                c = g * NBUF + b
                inb_wait(b, c)
                compute_b(b)

                @pl.when(c + NBUF < n_chunks)
                def _next(b=b, c=c):
                    inb_start(b, c + NBUF)
            return _

        lax.fori_loop(0, n_groups, main_b, None)

        for b in range(n_rem):
            c = n_groups * NBUF + b
            inb_wait(b, c)
            compute_b(b)

        pltpu.sync_copy(work_v, part_hbm.at[pl.ds(wid * n_nodes, n_nodes)])

    return fused


def _epilogue_body(x_ref, bias_ref, tau_ref, part_ref, o_ref):
    summed = jnp.sum(part_ref[...], axis=0)
    x = x_ref[...]
    o_ref[...] = x + DT * ((-x + bias_ref[...] + summed) / tau_ref[...])


def kernel(x, source_indices, target_indices, sign, syn_count, syn_strength,
           bias, time_const):
    n_nodes = x.shape[0]
    n_edges = source_indices.shape[0]

    fused = _build_fused(n_nodes, n_edges)
    partials = fused(x, source_indices.astype(jnp.int32), sign, syn_count,
                     syn_strength, target_indices.astype(jnp.int32))
    partials = partials.reshape(NW, n_nodes)

    x_new = pl.pallas_call(
        _epilogue_body,
        out_shape=jax.ShapeDtypeStruct((n_nodes,), jnp.float32),
    )(x, bias, time_const, partials)
    return x_new
